# baseline (speedup 1.0000x reference)
import jax
import jax.numpy as jnp
from jax import lax
from jax.experimental import pallas as pl
from jax.experimental.pallas import tpu as pltpu
from jax.experimental.pallas import tpu_sc as plsc

SEQ = 8192
E = 64
K = 8
SCALE = 16.0
CAP = float(E) / float(SEQ * SEQ * K)

NC = 2
NS = 16
L = 16
NW = NC * NS
ROWS_PER_W = SEQ // NW
NV = E // L


def _gating_body(x_hbm2, wout_hbm2, idx_hbm2, histp_hbm, colp_hbm,
                 x_v, wout_v, idx_v, hist_v, col_v):
    c = lax.axis_index("c")
    s = lax.axis_index("s")
    wid = s * NC + c
    base = wid * ROWS_PER_W

    pltpu.sync_copy(x_hbm2.at[pl.ds(base, ROWS_PER_W)], x_v)

    iota = lax.iota(jnp.int32, L)
    mask8 = iota < K
    ones16 = jnp.ones((L,), jnp.float32)
    perm8 = jnp.where(iota >= K, iota - K, iota)
    row_off = iota >> 3
    col_idx = iota & 7

    zeros16 = jnp.zeros((L,), jnp.float32)
    for cc in range(NV):
        hist_v[pl.ds(cc * L, L)] = zeros16
        col_v[pl.ds(cc * L, L)] = zeros16

    def allreduce(v, op):
        for sh in (8, 4, 2, 1):
            v = op(v, v.at[iota ^ sh].get(mode="promise_in_bounds"))
        return v

    def merge(ka, va, kb, vb):
        bk = kb.at[perm8].get(mode="promise_in_bounds")
        bv = vb.at[perm8].get(mode="promise_in_bounds")
        mk = jnp.where(mask8, ka, bk)
        mv = jnp.where(mask8, va, bv)
        return plsc.sort_key_val(mk, mv, descending=True)

    def one_row(r):
        v = [x_v[r, pl.ds(cc * L, L)] for cc in range(NV)]
        m = allreduce(jnp.maximum(jnp.maximum(v[0], v[1]),
                                  jnp.maximum(v[2], v[3])), jnp.maximum)
        e = [jnp.exp(vv - m) for vv in v]
        inv = 1.0 / allreduce(e[0] + e[1] + e[2] + e[3], jnp.add)
        g = [ee * inv for ee in e]
        for cc in range(NV):
            plsc.addupdate(col_v.at[pl.ds(cc * L, L)], g[cc])
        sk, sv = [], []
        for cc in range(NV):
            k2, v2 = plsc.sort_key_val(g[cc], iota + cc * L, descending=True)
            sk.append(k2)
            sv.append(v2)
        k01, v01 = merge(sk[0], sv[0], sk[1], sv[1])
        k23, v23 = merge(sk[2], sv[2], sk[3], sv[3])
        kf, vf = merge(k01, v01, k23, v23)
        plsc.addupdate_scatter(hist_v, [vf], ones16, mask=mask8)
        return kf, vf

    def pair_body(j, carry):
        k0, v0 = one_row(2 * j)
        k1, v1 = one_row(2 * j + 1)
        kp = jnp.where(mask8, k0, k1.at[perm8].get(mode="promise_in_bounds"))
        vp = jnp.where(mask8, v0, v1.at[perm8].get(mode="promise_in_bounds"))
        row_idx = row_off + 2 * j
        plsc.store_scatter(wout_v, [row_idx, col_idx], kp * SCALE)
        plsc.store_scatter(idx_v, [row_idx, col_idx], vp)
        return carry

    lax.fori_loop(0, ROWS_PER_W // 2, pair_body, 0)

    pltpu.sync_copy(wout_v, wout_hbm2.at[pl.ds(base, ROWS_PER_W)])
    pltpu.sync_copy(idx_v, idx_hbm2.at[pl.ds(base, ROWS_PER_W)])
    pltpu.sync_copy(hist_v, histp_hbm.at[wid])
    pltpu.sync_copy(col_v, colp_hbm.at[wid])


_gating = pl.kernel(
    _gating_body,
    out_type=[
        jax.ShapeDtypeStruct((SEQ, K), jnp.float32),
        jax.ShapeDtypeStruct((SEQ, K), jnp.int32),
        jax.ShapeDtypeStruct((NW, E), jnp.float32),
        jax.ShapeDtypeStruct((NW, E), jnp.float32),
    ],
    mesh=plsc.VectorSubcoreMesh(core_axis_name="c", subcore_axis_name="s"),
    compiler_params=pltpu.CompilerParams(needs_layout_passes=False,
                                         use_tc_tiling_on_sc=True),
    scratch_types=[
        pltpu.VMEM((ROWS_PER_W, E), jnp.float32),
        pltpu.VMEM((ROWS_PER_W, K), jnp.float32),
        pltpu.VMEM((ROWS_PER_W, K), jnp.int32),
        pltpu.VMEM((E,), jnp.float32),
        pltpu.VMEM((E,), jnp.float32),
    ],
)


def _loss_body(hp_ref, cp_ref, o_ref):
    hist = jnp.sum(hp_ref[...], axis=0)
    col = jnp.sum(cp_ref[...], axis=0)
    o_ref[0] = CAP * jnp.sum(hist * col)


_loss = pl.pallas_call(
    _loss_body,
    out_shape=jax.ShapeDtypeStruct((1,), jnp.float32),
    in_specs=[pl.BlockSpec(memory_space=pltpu.VMEM),
              pl.BlockSpec(memory_space=pltpu.VMEM)],
    out_specs=pl.BlockSpec(memory_space=pltpu.SMEM),
)


def kernel(input):
    x = input.astype(jnp.float32)
    wout, idx, histp, colp = _gating(x)
    loss = _loss(histp, colp)
    return (wout, loss, idx)


# speedup vs baseline: 1.1291x; 1.1291x over previous
import jax
import jax.numpy as jnp
from jax import lax
from jax.experimental import pallas as pl
from jax.experimental.pallas import tpu as pltpu
from jax.experimental.pallas import tpu_sc as plsc

SEQ = 8192
E = 64
K = 8
SCALE = 16.0
CAP = float(E) / float(SEQ * SEQ * K)

NC = 2
NS = 16
L = 16
NW = NC * NS
ROWS_PER_W = SEQ // NW
NV = E // L


def _gating_body(x_hbm2, wout_hbm2, idx_hbm2, histp_hbm, colp_hbm,
                 x_v, wout_v, idx_v, hist_v, col_v):
    c = lax.axis_index("c")
    s = lax.axis_index("s")
    wid = s * NC + c
    base = wid * ROWS_PER_W

    pltpu.sync_copy(x_hbm2.at[pl.ds(base, ROWS_PER_W)], x_v)

    iota = lax.iota(jnp.int32, L)
    mask8 = iota < K
    ones16 = jnp.ones((L,), jnp.float32)
    perm8 = jnp.where(iota >= K, iota - K, iota)
    row_off = iota >> 3
    col_idx = iota & 7
    lane0 = jnp.zeros((L,), jnp.int32)

    zeros16 = jnp.zeros((L,), jnp.float32)
    for cc in range(NV):
        hist_v[pl.ds(cc * L, L)] = zeros16

    def allreduce(v, op):
        for sh in (8, 4, 2, 1):
            v = op(v, v.at[iota ^ sh].get(mode="promise_in_bounds"))
        return v

    def merge(ka, va, kb, vb):
        bk = kb.at[perm8].get(mode="promise_in_bounds")
        bv = vb.at[perm8].get(mode="promise_in_bounds")
        mk = jnp.where(mask8, ka, bk)
        mv = jnp.where(mask8, va, bv)
        return plsc.sort_key_val(mk, mv, descending=True)

    def one_row(r, cols):
        v = [x_v[r, pl.ds(cc * L, L)] for cc in range(NV)]
        sk, sv = [], []
        for cc in range(NV):
            k2, v2 = plsc.sort_key_val(v[cc], iota + cc * L, descending=True)
            sk.append(k2)
            sv.append(v2)
        k01, v01 = merge(sk[0], sv[0], sk[1], sv[1])
        k23, v23 = merge(sk[2], sv[2], sk[3], sv[3])
        kf, vf = merge(k01, v01, k23, v23)
        m = kf.at[lane0].get(mode="promise_in_bounds")
        e = [jnp.exp(vv - m) for vv in v]
        inv = 1.0 / allreduce(e[0] + e[1] + e[2] + e[3], jnp.add)
        cols = tuple(cols[cc] + e[cc] * inv for cc in range(NV))
        w = jnp.exp(kf - m) * (inv * SCALE)
        plsc.addupdate_scatter(hist_v, [vf], ones16, mask=mask8)
        return w, vf, cols

    def pair_body(j, cols):
        w0, v0, cols = one_row(2 * j, cols)
        w1, v1, cols = one_row(2 * j + 1, cols)
        wp = jnp.where(mask8, w0, w1.at[perm8].get(mode="promise_in_bounds"))
        vp = jnp.where(mask8, v0, v1.at[perm8].get(mode="promise_in_bounds"))
        row_idx = row_off + 2 * j
        plsc.store_scatter(wout_v, [row_idx, col_idx], wp)
        plsc.store_scatter(idx_v, [row_idx, col_idx], vp)
        return cols

    cols = lax.fori_loop(0, ROWS_PER_W // 2, pair_body,
                         (zeros16, zeros16, zeros16, zeros16))
    for cc in range(NV):
        col_v[pl.ds(cc * L, L)] = cols[cc]

    pltpu.sync_copy(wout_v, wout_hbm2.at[pl.ds(base, ROWS_PER_W)])
    pltpu.sync_copy(idx_v, idx_hbm2.at[pl.ds(base, ROWS_PER_W)])
    pltpu.sync_copy(hist_v, histp_hbm.at[wid])
    pltpu.sync_copy(col_v, colp_hbm.at[wid])


_gating = pl.kernel(
    _gating_body,
    out_type=[
        jax.ShapeDtypeStruct((SEQ, K), jnp.float32),
        jax.ShapeDtypeStruct((SEQ, K), jnp.int32),
        jax.ShapeDtypeStruct((NW, E), jnp.float32),
        jax.ShapeDtypeStruct((NW, E), jnp.float32),
    ],
    mesh=plsc.VectorSubcoreMesh(core_axis_name="c", subcore_axis_name="s"),
    compiler_params=pltpu.CompilerParams(needs_layout_passes=False,
                                         use_tc_tiling_on_sc=True),
    scratch_types=[
        pltpu.VMEM((ROWS_PER_W, E), jnp.float32),
        pltpu.VMEM((ROWS_PER_W, K), jnp.float32),
        pltpu.VMEM((ROWS_PER_W, K), jnp.int32),
        pltpu.VMEM((E,), jnp.float32),
        pltpu.VMEM((E,), jnp.float32),
    ],
)


def _loss_body(hp_ref, cp_ref, o_ref):
    hist = jnp.sum(hp_ref[...], axis=0)
    col = jnp.sum(cp_ref[...], axis=0)
    o_ref[0] = CAP * jnp.sum(hist * col)


_loss = pl.pallas_call(
    _loss_body,
    out_shape=jax.ShapeDtypeStruct((1,), jnp.float32),
    in_specs=[pl.BlockSpec(memory_space=pltpu.VMEM),
              pl.BlockSpec(memory_space=pltpu.VMEM)],
    out_specs=pl.BlockSpec(memory_space=pltpu.SMEM),
)


def kernel(input):
    x = input.astype(jnp.float32)
    wout, idx, histp, colp = _gating(x)
    loss = _loss(histp, colp)
    return (wout, loss, idx)


# speedup vs baseline: 1.1748x; 1.0405x over previous
import jax
import jax.numpy as jnp
from jax import lax
from jax.experimental import pallas as pl
from jax.experimental.pallas import tpu as pltpu
from jax.experimental.pallas import tpu_sc as plsc

SEQ = 8192
E = 64
K = 8
SCALE = 16.0
CAP = float(E) / float(SEQ * SEQ * K)

NC = 2
NS = 16
L = 16
NW = NC * NS
ROWS_PER_W = SEQ // NW
NV = E // L


def _gating_body(x_hbm2, wout_hbm2, idx_hbm2, histp_hbm, colp_hbm,
                 x_v, wout_v, idx_v, hist_v, col_v):
    c = lax.axis_index("c")
    s = lax.axis_index("s")
    wid = s * NC + c
    base = wid * ROWS_PER_W

    pltpu.sync_copy(x_hbm2.at[pl.ds(base, ROWS_PER_W)], x_v)

    iota = lax.iota(jnp.int32, L)
    mask8 = iota < K
    ones16 = jnp.ones((L,), jnp.float32)
    perm8 = jnp.where(iota >= K, iota - K, iota)
    row_off = iota >> 3
    col_idx = iota & 7
    lane0 = jnp.zeros((L,), jnp.int32)

    zeros16 = jnp.zeros((L,), jnp.float32)
    for cc in range(NV):
        hist_v[pl.ds(cc * L, L)] = zeros16

    def allreduce(v, op):
        for sh in (8, 4, 2, 1):
            v = op(v, v.at[iota ^ sh].get(mode="promise_in_bounds"))
        return v

    def merge(ka, va, kb, vb):
        bk = kb.at[perm8].get(mode="promise_in_bounds")
        bv = vb.at[perm8].get(mode="promise_in_bounds")
        mk = jnp.where(mask8, ka, bk)
        mv = jnp.where(mask8, va, bv)
        return plsc.sort_key_val(mk, mv, descending=True)

    def one_row(r, cols):
        v = [x_v[r, pl.ds(cc * L, L)] for cc in range(NV)]
        sk, sv = [], []
        for cc in range(NV):
            k2, v2 = plsc.sort_key_val(v[cc], iota + cc * L, descending=True)
            sk.append(k2)
            sv.append(v2)
        k01, v01 = merge(sk[0], sv[0], sk[1], sv[1])
        k23, v23 = merge(sk[2], sv[2], sk[3], sv[3])
        kf, vf = merge(k01, v01, k23, v23)
        m = kf.at[lane0].get(mode="promise_in_bounds")
        e = [jnp.exp(vv - m) for vv in v]
        inv = 1.0 / allreduce(e[0] + e[1] + e[2] + e[3], jnp.add)
        cols = tuple(cols[cc] + e[cc] * inv for cc in range(NV))
        w = jnp.exp(kf - m) * (inv * SCALE)
        plsc.addupdate_scatter(hist_v, [vf], ones16, mask=mask8)
        return w, vf, cols

    def quad_body(j, cols):
        ws, vs = [], []
        for u in range(4):
            w, vv, cols = one_row(4 * j + u, cols)
            ws.append(w)
            vs.append(vv)
        for u in range(2):
            w0, w1 = ws[2 * u], ws[2 * u + 1]
            v0, v1 = vs[2 * u], vs[2 * u + 1]
            wp = jnp.where(mask8, w0,
                           w1.at[perm8].get(mode="promise_in_bounds"))
            vp = jnp.where(mask8, v0,
                           v1.at[perm8].get(mode="promise_in_bounds"))
            row_idx = row_off + 4 * j + 2 * u
            plsc.store_scatter(wout_v, [row_idx, col_idx], wp)
            plsc.store_scatter(idx_v, [row_idx, col_idx], vp)
        return cols

    cols = lax.fori_loop(0, ROWS_PER_W // 4, quad_body,
                         (zeros16, zeros16, zeros16, zeros16))
    for cc in range(NV):
        col_v[pl.ds(cc * L, L)] = cols[cc]

    pltpu.sync_copy(wout_v, wout_hbm2.at[pl.ds(base, ROWS_PER_W)])
    pltpu.sync_copy(idx_v, idx_hbm2.at[pl.ds(base, ROWS_PER_W)])
    pltpu.sync_copy(hist_v, histp_hbm.at[wid])
    pltpu.sync_copy(col_v, colp_hbm.at[wid])


_gating = pl.kernel(
    _gating_body,
    out_type=[
        jax.ShapeDtypeStruct((SEQ, K), jnp.float32),
        jax.ShapeDtypeStruct((SEQ, K), jnp.int32),
        jax.ShapeDtypeStruct((NW, E), jnp.float32),
        jax.ShapeDtypeStruct((NW, E), jnp.float32),
    ],
    mesh=plsc.VectorSubcoreMesh(core_axis_name="c", subcore_axis_name="s"),
    compiler_params=pltpu.CompilerParams(needs_layout_passes=False,
                                         use_tc_tiling_on_sc=True),
    scratch_types=[
        pltpu.VMEM((ROWS_PER_W, E), jnp.float32),
        pltpu.VMEM((ROWS_PER_W, K), jnp.float32),
        pltpu.VMEM((ROWS_PER_W, K), jnp.int32),
        pltpu.VMEM((E,), jnp.float32),
        pltpu.VMEM((E,), jnp.float32),
    ],
)


def _loss_body(hp_ref, cp_ref, o_ref):
    hist = jnp.sum(hp_ref[...], axis=0)
    col = jnp.sum(cp_ref[...], axis=0)
    o_ref[0] = CAP * jnp.sum(hist * col)


_loss = pl.pallas_call(
    _loss_body,
    out_shape=jax.ShapeDtypeStruct((1,), jnp.float32),
    in_specs=[pl.BlockSpec(memory_space=pltpu.VMEM),
              pl.BlockSpec(memory_space=pltpu.VMEM)],
    out_specs=pl.BlockSpec(memory_space=pltpu.SMEM),
)


def kernel(input):
    x = input.astype(jnp.float32)
    wout, idx, histp, colp = _gating(x)
    loss = _loss(histp, colp)
    return (wout, loss, idx)


# speedup vs baseline: 1.3511x; 1.1501x over previous
import jax
import jax.numpy as jnp
from jax import lax
from jax.experimental import pallas as pl
from jax.experimental.pallas import tpu as pltpu
from jax.experimental.pallas import tpu_sc as plsc

SEQ = 8192
E = 64
K = 8
SCALE = 16.0
CAP = float(E) / float(SEQ * SEQ * K)

NC = 2
NS = 16
L = 16
NW = NC * NS
ROWS_PER_W = SEQ // NW
NV = E // L


def _gating_body(x_hbm2, wout_hbm2, idx_hbm2, histp_hbm, colp_hbm,
                 x_v, wout_v, idx_v, hist_v, col_v):
    c = lax.axis_index("c")
    s = lax.axis_index("s")
    wid = s * NC + c
    base = wid * ROWS_PER_W

    pltpu.sync_copy(x_hbm2.at[pl.ds(base, ROWS_PER_W)], x_v)

    iota = lax.iota(jnp.int32, L)
    mask8 = iota < K
    ones16 = jnp.ones((L,), jnp.float32)
    perm8 = jnp.where(iota >= K, iota - K, iota)
    row_off = iota >> 3
    col_idx = iota & 7
    lane0 = jnp.zeros((L,), jnp.int32)

    zeros16 = jnp.zeros((L,), jnp.float32)
    for cc in range(NV):
        hist_v[pl.ds(cc * L, L)] = zeros16

    def allreduce(v, op):
        for sh in (8, 4, 2, 1):
            v = op(v, v.at[iota ^ sh].get(mode="promise_in_bounds"))
        return v

    def merge(ka, va, kb, vb):
        bk = kb.at[perm8].get(mode="promise_in_bounds")
        bv = vb.at[perm8].get(mode="promise_in_bounds")
        mk = jnp.where(mask8, ka, bk)
        mv = jnp.where(mask8, va, bv)
        return plsc.sort_key_val(mk, mv, descending=True)

    def one_row(r, cols):
        v = [x_v[r, pl.ds(cc * L, L)] for cc in range(NV)]
        sk, sv = [], []
        for cc in range(NV):
            k2, v2 = plsc.sort_key_val(v[cc], iota + cc * L, descending=True)
            sk.append(k2)
            sv.append(v2)
        k01, v01 = merge(sk[0], sv[0], sk[1], sv[1])
        k23, v23 = merge(sk[2], sv[2], sk[3], sv[3])
        kf, vf = merge(k01, v01, k23, v23)
        m = kf.at[lane0].get(mode="promise_in_bounds")
        e = [jnp.exp(vv - m) for vv in v]
        inv = 1.0 / allreduce(e[0] + e[1] + e[2] + e[3], jnp.add)
        cols = tuple(cols[cc] + e[cc] * inv for cc in range(NV))
        w = jnp.exp(kf - m) * (inv * SCALE)
        return w, vf, cols

    @plsc.parallel_loop(0, ROWS_PER_W // 2, unroll=2,
                        carry=(zeros16, zeros16, zeros16, zeros16))
    def pair_loop(j, cols):
        w0, v0, cols = one_row(2 * j, cols)
        w1, v1, cols = one_row(2 * j + 1, cols)
        wp = jnp.where(mask8, w0, w1.at[perm8].get(mode="promise_in_bounds"))
        vp = jnp.where(mask8, v0, v1.at[perm8].get(mode="promise_in_bounds"))
        row_idx = row_off + 2 * j
        plsc.store_scatter(wout_v, [row_idx, col_idx], wp)
        plsc.store_scatter(idx_v, [row_idx, col_idx], vp)
        return cols

    cols = pair_loop
    for cc in range(NV):
        col_v[pl.ds(cc * L, L)] = cols[cc]

    def hist_body(j, carry):
        vp = plsc.load_gather(idx_v, [row_off + 2 * j, col_idx])
        plsc.addupdate_scatter(hist_v, [vp], ones16)
        return carry

    lax.fori_loop(0, ROWS_PER_W // 2, hist_body, 0)

    pltpu.sync_copy(wout_v, wout_hbm2.at[pl.ds(base, ROWS_PER_W)])
    pltpu.sync_copy(idx_v, idx_hbm2.at[pl.ds(base, ROWS_PER_W)])
    pltpu.sync_copy(hist_v, histp_hbm.at[wid])
    pltpu.sync_copy(col_v, colp_hbm.at[wid])


_gating = pl.kernel(
    _gating_body,
    out_type=[
        jax.ShapeDtypeStruct((SEQ, K), jnp.float32),
        jax.ShapeDtypeStruct((SEQ, K), jnp.int32),
        jax.ShapeDtypeStruct((NW, E), jnp.float32),
        jax.ShapeDtypeStruct((NW, E), jnp.float32),
    ],
    mesh=plsc.VectorSubcoreMesh(core_axis_name="c", subcore_axis_name="s"),
    compiler_params=pltpu.CompilerParams(needs_layout_passes=False,
                                         use_tc_tiling_on_sc=True),
    scratch_types=[
        pltpu.VMEM((ROWS_PER_W, E), jnp.float32),
        pltpu.VMEM((ROWS_PER_W, K), jnp.float32),
        pltpu.VMEM((ROWS_PER_W, K), jnp.int32),
        pltpu.VMEM((E,), jnp.float32),
        pltpu.VMEM((E,), jnp.float32),
    ],
)


def _loss_body(hp_ref, cp_ref, o_ref):
    hist = jnp.sum(hp_ref[...], axis=0)
    col = jnp.sum(cp_ref[...], axis=0)
    o_ref[0] = CAP * jnp.sum(hist * col)


_loss = pl.pallas_call(
    _loss_body,
    out_shape=jax.ShapeDtypeStruct((1,), jnp.float32),
    in_specs=[pl.BlockSpec(memory_space=pltpu.VMEM),
              pl.BlockSpec(memory_space=pltpu.VMEM)],
    out_specs=pl.BlockSpec(memory_space=pltpu.SMEM),
)


def kernel(input):
    x = input.astype(jnp.float32)
    wout, idx, histp, colp = _gating(x)
    loss = _loss(histp, colp)
    return (wout, loss, idx)
